# Initial kernel scaffold; baseline (speedup 1.0000x reference)
import jax
import jax.numpy as jnp
from jax.experimental import pallas as pl


def kernel(mask_logits, ff_highres_features, log_strength, active_tile_indices):
    raise NotImplementedError("write your pallas kernel here")


# speedup vs baseline: 4.9417x; 4.9417x over previous
import jax
import jax.numpy as jnp
from jax.experimental import pallas as pl
from jax.experimental.pallas import tpu as pltpu

TS = 16
B, N, H, W = 2, 8, 384, 384
C = 96
K = 128
TH = H // TS
TW = W // TS


def _dense_body(idx_ref, ls_ref, mask_ref, ff_ref, out_ref):
    th = pl.program_id(1)
    x = ls_ref[0]
    strength = jnp.maximum(x, 0.0) + jnp.log(1.0 + jnp.exp(-jnp.abs(x)))
    ffb = ff_ref[0]
    sig = jnp.tanh(jnp.sum(ffb, axis=0) * (1.0 / C))
    tcol = th * TW + jax.lax.broadcasted_iota(jnp.int32, (TS, W), 1) // TS
    active = jnp.zeros((TS, W), dtype=jnp.bool_)
    for k in range(K):
        active = active | (tcol == idx_ref[0, 0, k])
    delta = jnp.where(active, strength * sig, 0.0)
    out_ref[0] = mask_ref[0] + delta[None, :, :]


def kernel(mask_logits, ff_highres_features, log_strength, active_tile_indices):
    idx = jnp.asarray(active_tile_indices, jnp.int32).reshape(B, 1, K)
    ls = jnp.asarray(log_strength, jnp.float32).reshape(1)
    grid = (B, TH)
    return pl.pallas_call(
        _dense_body,
        grid=grid,
        in_specs=[
            pl.BlockSpec((1, 1, K), lambda b, th: (b, 0, 0), memory_space=pltpu.SMEM),
            pl.BlockSpec(memory_space=pltpu.SMEM),
            pl.BlockSpec((1, N, TS, W), lambda b, th: (b, 0, th, 0)),
            pl.BlockSpec((1, C, TS, W), lambda b, th: (b, 0, th, 0)),
        ],
        out_specs=pl.BlockSpec((1, N, TS, W), lambda b, th: (b, 0, th, 0)),
        out_shape=jax.ShapeDtypeStruct((B, N, H, W), jnp.float32),
    )(idx, ls, mask_logits, ff_highres_features)
